# baseline (speedup 1.0000x reference)
import functools

import jax
import jax.numpy as jnp
from jax import lax
from jax.experimental import pallas as pl
from jax.experimental.pallas import tpu as pltpu
from jax.experimental.pallas import tpu_sc as plsc

EMBED = 128
KS = 16
NPAD = 1024
ROWS = 200
BIG = 1e10
NW = 32
RPW = 500
SLAB = 100
NCHUNK = NPAD // 16


def _keys_body(locsT_ref, rows_ref, keys_ref):
    ri = pl.program_id(1)
    xs = locsT_ref[0, 0:1, :]
    ys = locsT_ref[0, 1:2, :]
    xr = rows_ref[0, :, 0:1]
    yr = rows_ref[0, :, 1:2]
    dx = xr - xs
    dy = yr - ys
    sq = dx * dx + dy * dy
    rows_g = ri * ROWS + lax.broadcasted_iota(jnp.int32, (ROWS, NPAD), 0)
    cols = lax.broadcasted_iota(jnp.int32, (ROWS, NPAD), 1)
    sq = jnp.where(rows_g == cols, jnp.float32(BIG), sq)
    ikeys = lax.bitcast_convert_type(sq, jnp.int32)
    ikeys = (ikeys & jnp.int32(-1024)) | cols
    keys_ref[...] = lax.bitcast_convert_type(ikeys, jnp.float32)


def _sel_body(keys_hbm, valsq_hbm, dst_hbm, buf, osq, odst):
    wid = lax.axis_index("s") * 2 + lax.axis_index("c")
    base = wid * RPW

    def slab_body(p, _):
        row0 = base + p * SLAB
        pltpu.sync_copy(keys_hbm.at[pl.ds(row0 * NPAD, SLAB * NPAD)], buf)

        def row_body(r, _):
            level = []
            for c in range(NCHUNK):
                v = buf[pl.ds(r * NPAD + c * 16, 16)]
                level.append(plsc.sort_key_val(v, v)[0])
            while len(level) > 1:
                nxt = []
                for i in range(0, len(level), 2):
                    a = level[i]
                    bm = lax.rev(level[i + 1], (0,))
                    m = jnp.minimum(a, bm)
                    nxt.append(plsc.sort_key_val(m, m)[0])
                level = nxt
            best = level[0]
            ik = plsc.bitcast(best, jnp.int32)
            col = ik & jnp.int32(1023)
            sqv = plsc.bitcast(ik & jnp.int32(-1024), jnp.float32)
            grow = row0 + r
            boff = (grow // 1000) * 1000
            osq[pl.ds(r * KS, KS)] = sqv
            odst[pl.ds(r * KS, KS)] = col + boff
            return _

        lax.fori_loop(0, SLAB, row_body, None)
        pltpu.sync_copy(osq, valsq_hbm.at[pl.ds(row0 * KS, SLAB * KS)])
        pltpu.sync_copy(odst, dst_hbm.at[pl.ds(row0 * KS, SLAB * KS)])
        return _

    lax.fori_loop(0, RPW // SLAB, slab_body, None)


def _emb_body(v_ref, w_ref, b_ref, out_ref):
    vt = jnp.transpose(v_ref[...])
    vt = jnp.sqrt(jnp.maximum(vt, 1e-12))
    w = w_ref[...]
    bb = b_ref[...]
    for j in range(vt.shape[1]):
        out_ref[j * 128:(j + 1) * 128, :] = vt[:, j:j + 1] * w + bb


def kernel(locs, init_embedding, W, b):
    B, N, _ = locs.shape
    locsT = jnp.transpose(locs, (0, 2, 1))
    locsT = jnp.pad(locsT, ((0, 0), (0, 0), (0, NPAD - N)),
                    constant_values=1e4)
    keys = pl.pallas_call(
        _keys_body,
        grid=(B, N // ROWS),
        in_specs=[
            pl.BlockSpec((1, 2, NPAD), lambda bi, ri: (bi, 0, 0)),
            pl.BlockSpec((1, ROWS, 2), lambda bi, ri: (bi, ri, 0)),
        ],
        out_specs=pl.BlockSpec((ROWS, NPAD),
                               lambda bi, ri: (bi * (1000 // ROWS) + ri, 0)),
        out_shape=jax.ShapeDtypeStruct((B * N, NPAD), jnp.float32),
    )(locsT, locs)

    mesh = plsc.VectorSubcoreMesh(core_axis_name="c", subcore_axis_name="s")
    sel = functools.partial(
        pl.kernel,
        mesh=mesh,
        out_type=[
            jax.ShapeDtypeStruct((B * N * KS,), jnp.float32),
            jax.ShapeDtypeStruct((B * N * KS,), jnp.int32),
        ],
        scratch_types=[
            pltpu.VMEM((SLAB * NPAD,), jnp.float32),
            pltpu.VMEM((SLAB * KS,), jnp.float32),
            pltpu.VMEM((SLAB * KS,), jnp.int32),
        ],
        compiler_params=pltpu.CompilerParams(needs_layout_passes=False),
    )(_sel_body)
    valsq, dst = sel(keys.reshape(B * N * NPAD))

    E = B * N * KS
    VB = 16
    EB = 128 * VB
    edge_emb = pl.pallas_call(
        _emb_body,
        grid=(E // EB,),
        in_specs=[
            pl.BlockSpec((VB, 128), lambda i: (i, 0)),
            pl.BlockSpec((1, EMBED), lambda i: (0, 0)),
            pl.BlockSpec((1, EMBED), lambda i: (0, 0)),
        ],
        out_specs=pl.BlockSpec((EB, EMBED), lambda i: (i, 0)),
        out_shape=jax.ShapeDtypeStruct((E, EMBED), jnp.float32),
    )(valsq.reshape(E // 128, 128), W.reshape(1, EMBED),
      b.reshape(1, EMBED))

    offs = (jnp.arange(B) * N)[:, None]
    src = (jnp.repeat(jnp.arange(N), KS)[None, :] + offs).reshape(-1)
    edge_index = jnp.stack([src, dst])
    x = init_embedding.reshape(B * N, EMBED)
    return x, edge_index, edge_emb


# speedup vs baseline: 1.1804x; 1.1804x over previous
import functools

import jax
import jax.numpy as jnp
from jax import lax
from jax.experimental import pallas as pl
from jax.experimental.pallas import tpu as pltpu
from jax.experimental.pallas import tpu_sc as plsc

EMBED = 128
KS = 16
NPAD = 1024
ROWS = 200
BIG = 1e10
NW = 32
WROWS = 512
SLAB = 32
NSLAB = WROWS // SLAB
NCHUNK = NPAD // 16


def _keys_body(locsT_ref, rows_ref, keys_ref):
    ri = pl.program_id(1)
    xs = locsT_ref[0, 0:1, :]
    ys = locsT_ref[0, 1:2, :]
    xr = rows_ref[0, :, 0:1]
    yr = rows_ref[0, :, 1:2]
    dx = xr - xs
    dy = yr - ys
    sq = dx * dx + dy * dy
    rows_g = ri * ROWS + lax.broadcasted_iota(jnp.int32, (ROWS, NPAD), 0)
    cols = lax.broadcasted_iota(jnp.int32, (ROWS, NPAD), 1)
    sq = jnp.where(rows_g == cols, jnp.float32(BIG), sq)
    ikeys = lax.bitcast_convert_type(sq, jnp.int32)
    ikeys = (ikeys & jnp.int32(-1024)) | cols
    keys_ref[...] = lax.bitcast_convert_type(ikeys, jnp.float32)


def _sel_body(keys_hbm, valsq_hbm, dst_hbm, bufA, bufB, osq, odst,
              semA, semB):
    wid = lax.axis_index("s") * 2 + lax.axis_index("c")
    base = jnp.minimum((wid * 500) // 8 * 8, 16000 - WROWS)

    def start(slab_idx, buf, sem):
        row0 = pl.multiple_of(base + slab_idx * SLAB, 8)
        pltpu.async_copy(keys_hbm.at[pl.ds(row0, SLAB)], buf, sem)

    def drain(buf, sem):
        pltpu.make_async_copy(
            keys_hbm.at[pl.ds(0, SLAB)], buf, sem).wait()

    def process(slab_idx, buf):
        row0 = pl.multiple_of(base + slab_idx * SLAB, 8)

        def row_body(r, _):
            level = []
            for c in range(NCHUNK):
                v = buf[r, pl.ds(c * 16, 16)]
                level.append(plsc.sort_key_val(v, v)[0])
            while len(level) > 1:
                nxt = []
                for i in range(0, len(level), 2):
                    a = level[i]
                    bm = lax.rev(level[i + 1], (0,))
                    m = jnp.minimum(a, bm)
                    nxt.append(plsc.sort_key_val(m, m)[0])
                level = nxt
            best = level[0]
            ik = plsc.bitcast(best, jnp.int32)
            col = ik & jnp.int32(1023)
            sqv = plsc.bitcast(ik & jnp.int32(-1024), jnp.float32)
            grow = row0 + r
            boff = (grow // 1000) * 1000
            osq[r, :] = sqv
            odst[r, :] = col + boff
            return _

        lax.fori_loop(0, SLAB, row_body, None)
        pltpu.sync_copy(osq, valsq_hbm.at[pl.ds(row0, SLAB)])
        pltpu.sync_copy(odst, dst_hbm.at[pl.ds(row0, SLAB)])

    start(0, bufA, semA)

    def pair_body(q, _):
        start(2 * q + 1, bufB, semB)
        drain(bufA, semA)
        process(2 * q, bufA)
        start(2 * q + 2, bufA, semA)
        drain(bufB, semB)
        process(2 * q + 1, bufB)
        return _

    lax.fori_loop(0, NSLAB // 2 - 1, pair_body, None)
    start(NSLAB - 1, bufB, semB)
    drain(bufA, semA)
    process(NSLAB - 2, bufA)
    drain(bufB, semB)
    process(NSLAB - 1, bufB)


def _emb_body(v_ref, w_ref, b_ref, out_ref):
    vt = jnp.transpose(v_ref[...])
    vt = jnp.sqrt(jnp.maximum(vt, 1e-12))
    w = w_ref[...]
    bb = b_ref[...]
    for j in range(vt.shape[1]):
        out_ref[j * 128:(j + 1) * 128, :] = vt[:, j:j + 1] * w + bb


def kernel(locs, init_embedding, W, b):
    B, N, _ = locs.shape
    locsT = jnp.transpose(locs, (0, 2, 1))
    locsT = jnp.pad(locsT, ((0, 0), (0, 0), (0, NPAD - N)),
                    constant_values=1e4)
    keys = pl.pallas_call(
        _keys_body,
        grid=(B, N // ROWS),
        in_specs=[
            pl.BlockSpec((1, 2, NPAD), lambda bi, ri: (bi, 0, 0)),
            pl.BlockSpec((1, ROWS, 2), lambda bi, ri: (bi, ri, 0)),
        ],
        out_specs=pl.BlockSpec((ROWS, NPAD),
                               lambda bi, ri: (bi * (1000 // ROWS) + ri, 0)),
        out_shape=jax.ShapeDtypeStruct((B * N, NPAD), jnp.float32),
    )(locsT, locs)

    mesh = plsc.VectorSubcoreMesh(core_axis_name="c", subcore_axis_name="s")
    sel = functools.partial(
        pl.kernel,
        mesh=mesh,
        out_type=[
            jax.ShapeDtypeStruct((B * N, KS), jnp.float32),
            jax.ShapeDtypeStruct((B * N, KS), jnp.int32),
        ],
        scratch_types=[
            pltpu.VMEM((SLAB, NPAD), jnp.float32),
            pltpu.VMEM((SLAB, NPAD), jnp.float32),
            pltpu.VMEM((SLAB, KS), jnp.float32),
            pltpu.VMEM((SLAB, KS), jnp.int32),
            pltpu.SemaphoreType.DMA,
            pltpu.SemaphoreType.DMA,
        ],
        compiler_params=pltpu.CompilerParams(needs_layout_passes=False),
    )(_sel_body)
    valsq, dst = sel(keys)

    E = B * N * KS
    VB = 16
    EB = 128 * VB
    edge_emb = pl.pallas_call(
        _emb_body,
        grid=(E // EB,),
        in_specs=[
            pl.BlockSpec((VB, 128), lambda i: (i, 0)),
            pl.BlockSpec((1, EMBED), lambda i: (0, 0)),
            pl.BlockSpec((1, EMBED), lambda i: (0, 0)),
        ],
        out_specs=pl.BlockSpec((EB, EMBED), lambda i: (i, 0)),
        out_shape=jax.ShapeDtypeStruct((E, EMBED), jnp.float32),
    )(valsq.reshape(E // 128, 128), W.reshape(1, EMBED),
      b.reshape(1, EMBED))

    offs = (jnp.arange(B) * N)[:, None]
    src = (jnp.repeat(jnp.arange(N), KS)[None, :] + offs).reshape(-1)
    edge_index = jnp.stack([src, dst.reshape(-1)])
    x = init_embedding.reshape(B * N, EMBED)
    return x, edge_index, edge_emb


# speedup vs baseline: 1.2313x; 1.0431x over previous
import functools

import jax
import jax.numpy as jnp
from jax import lax
from jax.experimental import pallas as pl
from jax.experimental.pallas import tpu as pltpu
from jax.experimental.pallas import tpu_sc as plsc

EMBED = 128
KS = 16
NPAD = 1024
ROWS = 200
BIG = 1e10
NW = 32
WROWS = 512
SLAB = 32
NSLAB = WROWS // SLAB
NCHUNK = NPAD // 16


def _keys_body(locsT_ref, rows_ref, emb_ref, keys_ref, x_ref):
    ri = pl.program_id(1)
    x_ref[...] = emb_ref[0]
    xs = locsT_ref[0, 0:1, :]
    ys = locsT_ref[0, 1:2, :]
    xr = rows_ref[0, :, 0:1]
    yr = rows_ref[0, :, 1:2]
    dx = xr - xs
    dy = yr - ys
    sq = dx * dx + dy * dy
    rows_g = ri * ROWS + lax.broadcasted_iota(jnp.int32, (ROWS, NPAD), 0)
    cols = lax.broadcasted_iota(jnp.int32, (ROWS, NPAD), 1)
    sq = jnp.where(rows_g == cols, jnp.float32(BIG), sq)
    ikeys = lax.bitcast_convert_type(sq, jnp.int32)
    ikeys = (ikeys & jnp.int32(-1024)) | cols
    keys_ref[...] = lax.bitcast_convert_type(ikeys, jnp.float32)


def _sel_body(keys_hbm, valsq_hbm, dst_hbm, bufA, bufB, osq, odst,
              semA, semB):
    wid = lax.axis_index("s") * 2 + lax.axis_index("c")
    base = jnp.minimum((wid * 500) // 8 * 8, 16000 - WROWS)

    def start(slab_idx, buf, sem):
        row0 = pl.multiple_of(base + slab_idx * SLAB, 8)
        pltpu.async_copy(keys_hbm.at[pl.ds(row0, SLAB)], buf, sem)

    def drain(buf, sem):
        pltpu.make_async_copy(
            keys_hbm.at[pl.ds(0, SLAB)], buf, sem).wait()

    def process(slab_idx, buf):
        row0 = pl.multiple_of(base + slab_idx * SLAB, 8)

        def row_body(r, _):
            level = []
            for c in range(NCHUNK):
                v = buf[r, pl.ds(c * 16, 16)]
                level.append(plsc.sort_key_val(v, v)[0])
            while len(level) > 1:
                nxt = []
                for i in range(0, len(level), 2):
                    a = level[i]
                    bm = lax.rev(level[i + 1], (0,))
                    m = jnp.minimum(a, bm)
                    nxt.append(plsc.sort_key_val(m, m)[0])
                level = nxt
            best = level[0]
            ik = plsc.bitcast(best, jnp.int32)
            col = ik & jnp.int32(1023)
            sqv = plsc.bitcast(ik & jnp.int32(-1024), jnp.float32)
            grow = row0 + r
            boff = (grow // 1000) * 1000
            osq[pl.ds(r * KS, KS)] = sqv
            odst[pl.ds(r * KS, KS)] = col + boff
            return _

        lax.fori_loop(0, SLAB, row_body, None)
        pltpu.sync_copy(osq, valsq_hbm.at[pl.ds(row0 * KS, SLAB * KS)])
        pltpu.sync_copy(odst, dst_hbm.at[pl.ds(row0 * KS, SLAB * KS)])

    start(0, bufA, semA)

    def pair_body(q, _):
        start(2 * q + 1, bufB, semB)
        drain(bufA, semA)
        process(2 * q, bufA)
        start(2 * q + 2, bufA, semA)
        drain(bufB, semB)
        process(2 * q + 1, bufB)
        return _

    lax.fori_loop(0, NSLAB // 2 - 1, pair_body, None)
    start(NSLAB - 1, bufB, semB)
    drain(bufA, semA)
    process(NSLAB - 2, bufA)
    drain(bufB, semB)
    process(NSLAB - 1, bufB)


def _emb_body(v_ref, w_ref, b_ref, out_ref):
    vt = jnp.transpose(v_ref[...])
    vt = jnp.sqrt(jnp.maximum(vt, 1e-12))
    w = w_ref[...]
    bb = b_ref[...]
    for j in range(vt.shape[1]):
        out_ref[j * 128:(j + 1) * 128, :] = vt[:, j:j + 1] * w + bb


def kernel(locs, init_embedding, W, b):
    B, N, _ = locs.shape
    locsT = jnp.transpose(locs, (0, 2, 1))
    locsT = jnp.pad(locsT, ((0, 0), (0, 0), (0, NPAD - N)),
                    constant_values=1e4)
    keys, x = pl.pallas_call(
        _keys_body,
        grid=(B, N // ROWS),
        in_specs=[
            pl.BlockSpec((1, 2, NPAD), lambda bi, ri: (bi, 0, 0)),
            pl.BlockSpec((1, ROWS, 2), lambda bi, ri: (bi, ri, 0)),
            pl.BlockSpec((1, ROWS, EMBED), lambda bi, ri: (bi, ri, 0)),
        ],
        out_specs=[
            pl.BlockSpec((ROWS, NPAD),
                         lambda bi, ri: (bi * (1000 // ROWS) + ri, 0)),
            pl.BlockSpec((ROWS, EMBED),
                         lambda bi, ri: (bi * (1000 // ROWS) + ri, 0)),
        ],
        out_shape=[
            jax.ShapeDtypeStruct((B * N, NPAD), jnp.float32),
            jax.ShapeDtypeStruct((B * N, EMBED), jnp.float32),
        ],
    )(locsT, locs, init_embedding)

    mesh = plsc.VectorSubcoreMesh(core_axis_name="c", subcore_axis_name="s")
    sel = functools.partial(
        pl.kernel,
        mesh=mesh,
        out_type=[
            jax.ShapeDtypeStruct((B * N * KS,), jnp.float32),
            jax.ShapeDtypeStruct((B * N * KS,), jnp.int32),
        ],
        scratch_types=[
            pltpu.VMEM((SLAB, NPAD), jnp.float32),
            pltpu.VMEM((SLAB, NPAD), jnp.float32),
            pltpu.VMEM((SLAB * KS,), jnp.float32),
            pltpu.VMEM((SLAB * KS,), jnp.int32),
            pltpu.SemaphoreType.DMA,
            pltpu.SemaphoreType.DMA,
        ],
        compiler_params=pltpu.CompilerParams(needs_layout_passes=False),
    )(_sel_body)
    valsq, dst = sel(keys)

    E = B * N * KS
    VB = 16
    EB = 128 * VB
    edge_emb = pl.pallas_call(
        _emb_body,
        grid=(E // EB,),
        in_specs=[
            pl.BlockSpec((VB, 128), lambda i: (i, 0)),
            pl.BlockSpec((1, EMBED), lambda i: (0, 0)),
            pl.BlockSpec((1, EMBED), lambda i: (0, 0)),
        ],
        out_specs=pl.BlockSpec((EB, EMBED), lambda i: (i, 0)),
        out_shape=jax.ShapeDtypeStruct((E, EMBED), jnp.float32),
    )(valsq.reshape(E // 128, 128), W.reshape(1, EMBED),
      b.reshape(1, EMBED))

    offs = (jnp.arange(B) * N)[:, None]
    src = (jnp.repeat(jnp.arange(N), KS)[None, :] + offs).reshape(-1)
    edge_index = jnp.stack([src, dst])
    x = init_embedding.reshape(B * N, EMBED)
    return x, edge_index, edge_emb


# speedup vs baseline: 1.6182x; 1.3143x over previous
import functools

import jax
import jax.numpy as jnp
from jax import lax
from jax.experimental import pallas as pl
from jax.experimental.pallas import tpu as pltpu
from jax.experimental.pallas import tpu_sc as plsc

EMBED = 128
KS = 16
NPAD = 1024
ROWS = 200
BIG = 1e10
NW = 32
SLAB = 32
NCHUNK = NPAD // 16
HB = 8
HROWS = HB * 1000
WROWS = 256
NSLAB = WROWS // SLAB


def _keys_body(locsT_ref, rows_ref, keys_ref):
    ri = pl.program_id(1)
    xs = locsT_ref[0, 0:1, :]
    ys = locsT_ref[0, 1:2, :]
    xr = rows_ref[0, :, 0:1]
    yr = rows_ref[0, :, 1:2]
    dx = xr - xs
    dy = yr - ys
    sq = dx * dx + dy * dy
    rows_g = ri * ROWS + lax.broadcasted_iota(jnp.int32, (ROWS, NPAD), 0)
    cols = lax.broadcasted_iota(jnp.int32, (ROWS, NPAD), 1)
    sq = jnp.where(rows_g == cols, jnp.float32(BIG), sq)
    ikeys = lax.bitcast_convert_type(sq, jnp.int32)
    ikeys = (ikeys & jnp.int32(-1024)) | cols
    keys_ref[...] = lax.bitcast_convert_type(ikeys, jnp.float32)


def _sel_body(keys_hbm, valsq_hbm, dst_hbm, bufA, bufB, osq, odst,
              semA, semB):
    wid = lax.axis_index("s") * 2 + lax.axis_index("c")
    base = jnp.minimum((wid * (HROWS // NW)) // 8 * 8, HROWS - WROWS)

    def start(slab_idx, buf, sem):
        row0 = pl.multiple_of(base + slab_idx * SLAB, 8)
        pltpu.async_copy(keys_hbm.at[pl.ds(row0, SLAB)], buf, sem)

    def drain(buf, sem):
        pltpu.make_async_copy(
            keys_hbm.at[pl.ds(0, SLAB)], buf, sem).wait()

    def process(slab_idx, buf):
        row0 = pl.multiple_of(base + slab_idx * SLAB, 8)

        def row_body(r, _):
            level = []
            for c in range(NCHUNK):
                v = buf[r, pl.ds(c * 16, 16)]
                level.append(plsc.sort_key_val(v, v)[0])
            while len(level) > 1:
                nxt = []
                for i in range(0, len(level), 2):
                    a = level[i]
                    bm = lax.rev(level[i + 1], (0,))
                    m = jnp.minimum(a, bm)
                    nxt.append(plsc.sort_key_val(m, m)[0])
                level = nxt
            best = level[0]
            ik = plsc.bitcast(best, jnp.int32)
            col = ik & jnp.int32(1023)
            sqv = plsc.bitcast(ik & jnp.int32(-1024), jnp.float32)
            grow = row0 + r
            boff = (grow // 1000) * 1000
            osq[pl.ds(r * KS, KS)] = sqv
            odst[pl.ds(r * KS, KS)] = col + boff
            return _

        lax.fori_loop(0, SLAB, row_body, None)
        pltpu.sync_copy(osq, valsq_hbm.at[pl.ds(row0 * KS, SLAB * KS)])
        pltpu.sync_copy(odst, dst_hbm.at[pl.ds(row0 * KS, SLAB * KS)])

    start(0, bufA, semA)

    def pair_body(q, _):
        start(2 * q + 1, bufB, semB)
        drain(bufA, semA)
        process(2 * q, bufA)
        start(2 * q + 2, bufA, semA)
        drain(bufB, semB)
        process(2 * q + 1, bufB)
        return _

    lax.fori_loop(0, NSLAB // 2 - 1, pair_body, None)
    start(NSLAB - 1, bufB, semB)
    drain(bufA, semA)
    process(NSLAB - 2, bufA)
    drain(bufB, semB)
    process(NSLAB - 1, bufB)


def _emb_body(v1_ref, v2_ref, w_ref, b_ref, out_ref):
    half1 = pl.program_id(0) < (HROWS * KS // (128 * VB))
    v = jnp.where(half1, v1_ref[...], v2_ref[...])
    vt = jnp.transpose(v)
    vt = jnp.sqrt(jnp.maximum(vt, 1e-12))
    w = w_ref[...]
    bb = b_ref[...]
    for j in range(vt.shape[1]):
        out_ref[j * 128:(j + 1) * 128, :] = vt[:, j:j + 1] * w + bb


def _tc_keys_half(locsT_h, locs_h):
    return pl.pallas_call(
        _keys_body,
        grid=(HB, 1000 // ROWS),
        in_specs=[
            pl.BlockSpec((1, 2, NPAD), lambda bi, ri: (bi, 0, 0)),
            pl.BlockSpec((1, ROWS, 2), lambda bi, ri: (bi, ri, 0)),
        ],
        out_specs=pl.BlockSpec((ROWS, NPAD),
                               lambda bi, ri: (bi * (1000 // ROWS) + ri, 0)),
        out_shape=jax.ShapeDtypeStruct((HROWS, NPAD), jnp.float32),
    )(locsT_h, locs_h)


def _sc_sel_half(keys_h):
    mesh = plsc.VectorSubcoreMesh(core_axis_name="c", subcore_axis_name="s")
    sel = functools.partial(
        pl.kernel,
        mesh=mesh,
        out_type=[
            jax.ShapeDtypeStruct((HROWS * KS,), jnp.float32),
            jax.ShapeDtypeStruct((HROWS * KS,), jnp.int32),
        ],
        scratch_types=[
            pltpu.VMEM((SLAB, NPAD), jnp.float32),
            pltpu.VMEM((SLAB, NPAD), jnp.float32),
            pltpu.VMEM((SLAB * KS,), jnp.float32),
            pltpu.VMEM((SLAB * KS,), jnp.int32),
            pltpu.SemaphoreType.DMA,
            pltpu.SemaphoreType.DMA,
        ],
        compiler_params=pltpu.CompilerParams(needs_layout_passes=False),
    )(_sel_body)
    return sel(keys_h)


VB = 40
HSTEPS = HROWS * KS // (128 * VB)


def _tc_emb(valsq1, valsq2, W, b):
    E = 2 * HROWS * KS
    EB = 128 * VB
    EH = HROWS * KS
    return pl.pallas_call(
        _emb_body,
        grid=(E // EB,),
        in_specs=[
            pl.BlockSpec((VB, 128), lambda i: (jnp.minimum(i, HSTEPS - 1), 0)),
            pl.BlockSpec((VB, 128),
                         (lambda i: (jnp.maximum(i - HSTEPS, 0), 0))),
            pl.BlockSpec((1, EMBED), lambda i: (0, 0)),
            pl.BlockSpec((1, EMBED), lambda i: (0, 0)),
        ],
        out_specs=pl.BlockSpec((EB, EMBED), lambda i: (i, 0)),
        out_shape=jax.ShapeDtypeStruct((E, EMBED), jnp.float32),
    )(valsq1.reshape(EH // 128, 128), valsq2.reshape(EH // 128, 128),
      W.reshape(1, EMBED), b.reshape(1, EMBED))


def kernel(locs, init_embedding, W, b):
    B, N, _ = locs.shape
    locsT = jnp.transpose(locs, (0, 2, 1))
    locsT = jnp.pad(locsT, ((0, 0), (0, 0), (0, NPAD - N)),
                    constant_values=1e4)

    keys1 = _tc_keys_half(locsT[:HB], locs[:HB])
    valsq1, dst1 = _sc_sel_half(keys1)
    keys2 = _tc_keys_half(locsT[HB:], locs[HB:])
    valsq2, dst2 = _sc_sel_half(keys2)
    edge_emb = _tc_emb(valsq1, valsq2, W, b)

    offs = (jnp.arange(B) * N)[:, None]
    src = (jnp.repeat(jnp.arange(N), KS)[None, :] + offs).reshape(-1)
    edge_index = jnp.stack([src, jnp.concatenate([dst1, dst2 + HROWS])])
    x = init_embedding.reshape(B * N, EMBED)
    return x, edge_index, edge_emb


# speedup vs baseline: 1.6622x; 1.0272x over previous
import functools

import jax
import jax.numpy as jnp
from jax import lax
from jax.experimental import pallas as pl
from jax.experimental.pallas import tpu as pltpu
from jax.experimental.pallas import tpu_sc as plsc

EMBED = 128
KS = 16
NPAD = 1024
ROWS = 200
BIG = 1e10
NW = 32
SLAB = 32
NCHUNK = NPAD // 16
HB = 8
HROWS = HB * 1000
WROWS = 256
NSLAB = WROWS // SLAB


def _keys_body(locsT_ref, rows_ref, keys_ref):
    ri = pl.program_id(1)
    xs = locsT_ref[0, 0:1, :]
    ys = locsT_ref[0, 1:2, :]
    xr = rows_ref[0, :, 0:1]
    yr = rows_ref[0, :, 1:2]
    dx = xr - xs
    dy = yr - ys
    sq = dx * dx + dy * dy
    rows_g = ri * ROWS + lax.broadcasted_iota(jnp.int32, (ROWS, NPAD), 0)
    cols = lax.broadcasted_iota(jnp.int32, (ROWS, NPAD), 1)
    sq = jnp.where(rows_g == cols, jnp.float32(BIG), sq)
    ikeys = lax.bitcast_convert_type(sq, jnp.int32)
    ikeys = (ikeys & jnp.int32(-1024)) | cols
    keys_ref[...] = lax.bitcast_convert_type(ikeys, jnp.float32)


def _sel_body(keys_hbm, valsq_hbm, dst_hbm, bufA, bufB, osq, odst,
              semA, semB):
    wid = lax.axis_index("s") * 2 + lax.axis_index("c")
    base = jnp.minimum((wid * (HROWS // NW)) // 8 * 8, HROWS - WROWS)

    def start(slab_idx, buf, sem):
        row0 = pl.multiple_of(base + slab_idx * SLAB, 8)
        pltpu.async_copy(keys_hbm.at[pl.ds(row0, SLAB)], buf, sem)

    def drain(buf, sem):
        pltpu.make_async_copy(
            keys_hbm.at[pl.ds(0, SLAB)], buf, sem).wait()

    def process(slab_idx, buf):
        row0 = pl.multiple_of(base + slab_idx * SLAB, 8)

        def row_body(r):
            level = []
            for c in range(NCHUNK):
                v = buf[r, pl.ds(c * 16, 16)]
                level.append(plsc.sort_key_val(v, v)[0])
            while len(level) > 1:
                nxt = []
                for i in range(0, len(level), 2):
                    a = level[i]
                    bm = lax.rev(level[i + 1], (0,))
                    m = jnp.minimum(a, bm)
                    nxt.append(plsc.sort_key_val(m, m)[0])
                level = nxt
            best = level[0]
            ik = plsc.bitcast(best, jnp.int32)
            col = ik & jnp.int32(1023)
            sqv = plsc.bitcast(ik & jnp.int32(-1024), jnp.float32)
            grow = row0 + r
            boff = (grow // 1000) * 1000
            osq[pl.ds(r * KS, KS)] = sqv
            odst[pl.ds(r * KS, KS)] = col + boff

        plsc.parallel_loop(0, SLAB, 1, unroll=2)(row_body)
        pltpu.sync_copy(osq, valsq_hbm.at[pl.ds(row0 * KS, SLAB * KS)])
        pltpu.sync_copy(odst, dst_hbm.at[pl.ds(row0 * KS, SLAB * KS)])

    start(0, bufA, semA)

    def pair_body(q, _):
        start(2 * q + 1, bufB, semB)
        drain(bufA, semA)
        process(2 * q, bufA)
        start(2 * q + 2, bufA, semA)
        drain(bufB, semB)
        process(2 * q + 1, bufB)
        return _

    lax.fori_loop(0, NSLAB // 2 - 1, pair_body, None)
    start(NSLAB - 1, bufB, semB)
    drain(bufA, semA)
    process(NSLAB - 2, bufA)
    drain(bufB, semB)
    process(NSLAB - 1, bufB)


def _emb_body(v1_ref, v2_ref, w_ref, b_ref, out_ref):
    half1 = pl.program_id(0) < (HROWS * KS // (128 * VB))
    v = jnp.where(half1, v1_ref[...], v2_ref[...])
    vt = jnp.transpose(v)
    vt = jnp.sqrt(jnp.maximum(vt, 1e-12))
    w = w_ref[...]
    bb = b_ref[...]
    for j in range(vt.shape[1]):
        out_ref[j * 128:(j + 1) * 128, :] = vt[:, j:j + 1] * w + bb


def _tc_keys_half(locsT_h, locs_h):
    return pl.pallas_call(
        _keys_body,
        grid=(HB, 1000 // ROWS),
        in_specs=[
            pl.BlockSpec((1, 2, NPAD), lambda bi, ri: (bi, 0, 0)),
            pl.BlockSpec((1, ROWS, 2), lambda bi, ri: (bi, ri, 0)),
        ],
        out_specs=pl.BlockSpec((ROWS, NPAD),
                               lambda bi, ri: (bi * (1000 // ROWS) + ri, 0)),
        out_shape=jax.ShapeDtypeStruct((HROWS, NPAD), jnp.float32),
    )(locsT_h, locs_h)


def _sc_sel_half(keys_h):
    mesh = plsc.VectorSubcoreMesh(core_axis_name="c", subcore_axis_name="s")
    sel = functools.partial(
        pl.kernel,
        mesh=mesh,
        out_type=[
            jax.ShapeDtypeStruct((HROWS * KS,), jnp.float32),
            jax.ShapeDtypeStruct((HROWS * KS,), jnp.int32),
        ],
        scratch_types=[
            pltpu.VMEM((SLAB, NPAD), jnp.float32),
            pltpu.VMEM((SLAB, NPAD), jnp.float32),
            pltpu.VMEM((SLAB * KS,), jnp.float32),
            pltpu.VMEM((SLAB * KS,), jnp.int32),
            pltpu.SemaphoreType.DMA,
            pltpu.SemaphoreType.DMA,
        ],
        compiler_params=pltpu.CompilerParams(needs_layout_passes=False),
    )(_sel_body)
    return sel(keys_h)


VB = 40
HSTEPS = HROWS * KS // (128 * VB)


def _tc_emb(valsq1, valsq2, W, b):
    E = 2 * HROWS * KS
    EB = 128 * VB
    EH = HROWS * KS
    return pl.pallas_call(
        _emb_body,
        grid=(E // EB,),
        in_specs=[
            pl.BlockSpec((VB, 128), lambda i: (jnp.minimum(i, HSTEPS - 1), 0)),
            pl.BlockSpec((VB, 128),
                         (lambda i: (jnp.maximum(i - HSTEPS, 0), 0))),
            pl.BlockSpec((1, EMBED), lambda i: (0, 0)),
            pl.BlockSpec((1, EMBED), lambda i: (0, 0)),
        ],
        out_specs=pl.BlockSpec((EB, EMBED), lambda i: (i, 0)),
        out_shape=jax.ShapeDtypeStruct((E, EMBED), jnp.float32),
    )(valsq1.reshape(EH // 128, 128), valsq2.reshape(EH // 128, 128),
      W.reshape(1, EMBED), b.reshape(1, EMBED))


def kernel(locs, init_embedding, W, b):
    B, N, _ = locs.shape
    locsT = jnp.transpose(locs, (0, 2, 1))
    locsT = jnp.pad(locsT, ((0, 0), (0, 0), (0, NPAD - N)),
                    constant_values=1e4)

    keys1 = _tc_keys_half(locsT[:HB], locs[:HB])
    valsq1, dst1 = _sc_sel_half(keys1)
    keys2 = _tc_keys_half(locsT[HB:], locs[HB:])
    valsq2, dst2 = _sc_sel_half(keys2)
    edge_emb = _tc_emb(valsq1, valsq2, W, b)

    offs = (jnp.arange(B) * N)[:, None]
    src = (jnp.repeat(jnp.arange(N), KS)[None, :] + offs).reshape(-1)
    edge_index = jnp.stack([src, jnp.concatenate([dst1, dst2 + HROWS])])
    x = init_embedding.reshape(B * N, EMBED)
    return x, edge_index, edge_emb


# speedup vs baseline: 1.7958x; 1.0804x over previous
import functools

import jax
import jax.numpy as jnp
from jax import lax
from jax.experimental import pallas as pl
from jax.experimental.pallas import tpu as pltpu
from jax.experimental.pallas import tpu_sc as plsc

EMBED = 128
KS = 16
NPAD = 1024
ROWS = 200
BIG = 1e10
NW = 32
SLAB = 32
NCHUNK = NPAD // 16
HB = 8
HROWS = HB * 1000
WROWS = 256
NSLAB = WROWS // SLAB


def _keys_body(locsT_ref, rows_ref, keys_ref):
    ri = pl.program_id(1)
    xs = locsT_ref[0, 0:1, :]
    ys = locsT_ref[0, 1:2, :]
    xr = rows_ref[0, :, 0:1]
    yr = rows_ref[0, :, 1:2]
    dx = xr - xs
    dy = yr - ys
    sq = dx * dx + dy * dy
    rows_g = ri * ROWS + lax.broadcasted_iota(jnp.int32, (ROWS, NPAD), 0)
    cols = lax.broadcasted_iota(jnp.int32, (ROWS, NPAD), 1)
    sq = jnp.where(rows_g == cols, jnp.float32(BIG), sq)
    ikeys = lax.bitcast_convert_type(sq, jnp.int32)
    ikeys = (ikeys & jnp.int32(-1024)) | cols
    keys_ref[...] = lax.bitcast_convert_type(ikeys, jnp.float32)


def _sel_body(keys_hbm, valsq_hbm, dst_hbm, bufA, bufB, osq, odst,
              semA, semB):
    wid = lax.axis_index("s") * 2 + lax.axis_index("c")
    base = jnp.minimum((wid * (HROWS // NW)) // 8 * 8, HROWS - WROWS)

    def start(slab_idx, buf, sem):
        row0 = pl.multiple_of(base + slab_idx * SLAB, 8)
        pltpu.async_copy(keys_hbm.at[pl.ds(row0, SLAB)], buf, sem)

    def drain(buf, sem):
        pltpu.make_async_copy(
            keys_hbm.at[pl.ds(0, SLAB)], buf, sem).wait()

    def process(slab_idx, buf):
        row0 = pl.multiple_of(base + slab_idx * SLAB, 8)

        def row_body(r):
            level = []
            for c in range(NCHUNK):
                v = buf[r, pl.ds(c * 16, 16)]
                level.append(plsc.sort_key_val(v, v)[0])
            while len(level) > 1:
                nxt = []
                for i in range(0, len(level), 2):
                    a = level[i]
                    bm = lax.rev(level[i + 1], (0,))
                    m = jnp.minimum(a, bm)
                    nxt.append(plsc.sort_key_val(m, m)[0])
                level = nxt
            best = level[0]
            ik = plsc.bitcast(best, jnp.int32)
            col = ik & jnp.int32(1023)
            sqv = plsc.bitcast(ik & jnp.int32(-1024), jnp.float32)
            grow = row0 + r
            boff = (grow // 1000) * 1000
            osq[pl.ds(r * KS, KS)] = sqv
            odst[pl.ds(r * KS, KS)] = col + boff

        plsc.parallel_loop(0, SLAB, 1, unroll=2)(row_body)
        pltpu.sync_copy(osq, valsq_hbm.at[pl.ds(row0 * KS, SLAB * KS)])
        pltpu.sync_copy(odst, dst_hbm.at[pl.ds(row0 * KS, SLAB * KS)])

    start(0, bufA, semA)

    def pair_body(q, _):
        start(2 * q + 1, bufB, semB)
        drain(bufA, semA)
        process(2 * q, bufA)
        start(2 * q + 2, bufA, semA)
        drain(bufB, semB)
        process(2 * q + 1, bufB)
        return _

    lax.fori_loop(0, NSLAB // 2 - 1, pair_body, None)
    start(NSLAB - 1, bufB, semB)
    drain(bufA, semA)
    process(NSLAB - 2, bufA)
    drain(bufB, semB)
    process(NSLAB - 1, bufB)


def _emb_body(v1_ref, v2_ref, w_ref, b_ref, out_ref):
    half1 = pl.program_id(0) < (HROWS * KS // (128 * VB))
    v = jnp.where(half1, v1_ref[...], v2_ref[...])
    vt = jnp.transpose(v)
    vt = jnp.sqrt(jnp.maximum(vt, 1e-12))
    w = w_ref[...]
    bb = b_ref[...]
    for j in range(vt.shape[1]):
        out_ref[j * 128:(j + 1) * 128, :] = vt[:, j:j + 1] * w + bb


def _tc_keys_half(locsT_h, locs_h):
    return pl.pallas_call(
        _keys_body,
        grid=(HB, 1000 // ROWS),
        in_specs=[
            pl.BlockSpec((1, 2, NPAD), lambda bi, ri: (bi, 0, 0)),
            pl.BlockSpec((1, ROWS, 2), lambda bi, ri: (bi, ri, 0)),
        ],
        out_specs=pl.BlockSpec((ROWS, NPAD),
                               lambda bi, ri: (bi * (1000 // ROWS) + ri, 0)),
        out_shape=jax.ShapeDtypeStruct((HROWS, NPAD), jnp.float32),
    )(locsT_h, locs_h)


def _sc_sel_half(keys_h):
    mesh = plsc.VectorSubcoreMesh(core_axis_name="c", subcore_axis_name="s")
    sel = functools.partial(
        pl.kernel,
        mesh=mesh,
        out_type=[
            jax.ShapeDtypeStruct((HROWS * KS,), jnp.float32),
            jax.ShapeDtypeStruct((HROWS * KS,), jnp.int32),
        ],
        scratch_types=[
            pltpu.VMEM((SLAB, NPAD), jnp.float32),
            pltpu.VMEM((SLAB, NPAD), jnp.float32),
            pltpu.VMEM((SLAB * KS,), jnp.float32),
            pltpu.VMEM((SLAB * KS,), jnp.int32),
            pltpu.SemaphoreType.DMA,
            pltpu.SemaphoreType.DMA,
        ],
        compiler_params=pltpu.CompilerParams(needs_layout_passes=False),
    )(_sel_body)
    return sel(keys_h)


VB = 200
HSTEPS = HROWS * KS // (128 * VB)


def _tc_emb(valsq1, valsq2, W, b):
    E = 2 * HROWS * KS
    EB = 128 * VB
    EH = HROWS * KS
    return pl.pallas_call(
        _emb_body,
        grid=(E // EB,),
        in_specs=[
            pl.BlockSpec((VB, 128), lambda i: (jnp.minimum(i, HSTEPS - 1), 0)),
            pl.BlockSpec((VB, 128),
                         (lambda i: (jnp.maximum(i - HSTEPS, 0), 0))),
            pl.BlockSpec((1, EMBED), lambda i: (0, 0)),
            pl.BlockSpec((1, EMBED), lambda i: (0, 0)),
        ],
        out_specs=pl.BlockSpec((EB, EMBED), lambda i: (i, 0)),
        out_shape=jax.ShapeDtypeStruct((E, EMBED), jnp.float32),
    )(valsq1.reshape(EH // 128, 128), valsq2.reshape(EH // 128, 128),
      W.reshape(1, EMBED), b.reshape(1, EMBED))


def kernel(locs, init_embedding, W, b):
    B, N, _ = locs.shape
    locsT = jnp.transpose(locs, (0, 2, 1))
    locsT = jnp.pad(locsT, ((0, 0), (0, 0), (0, NPAD - N)),
                    constant_values=1e4)

    keys1 = _tc_keys_half(locsT[:HB], locs[:HB])
    valsq1, dst1 = _sc_sel_half(keys1)
    keys2 = _tc_keys_half(locsT[HB:], locs[HB:])
    valsq2, dst2 = _sc_sel_half(keys2)
    edge_emb = _tc_emb(valsq1, valsq2, W, b)

    offs = (jnp.arange(B) * N)[:, None]
    src = (jnp.repeat(jnp.arange(N), KS)[None, :] + offs).reshape(-1)
    edge_index = jnp.stack([src, jnp.concatenate([dst1, dst2 + HROWS])])
    x = init_embedding.reshape(B * N, EMBED)
    return x, edge_index, edge_emb


# speedup vs baseline: 1.9437x; 1.0824x over previous
import functools

import jax
import jax.numpy as jnp
from jax import lax
from jax.experimental import pallas as pl
from jax.experimental.pallas import tpu as pltpu
from jax.experimental.pallas import tpu_sc as plsc

EMBED = 128
KS = 16
NPAD = 1024
ROWS = 1000
BIG = 1e10
NW = 32
SLAB = 32
NCHUNK = NPAD // 16
HB = 8
HROWS = HB * 1000
WROWS = 256
NSLAB = WROWS // SLAB


def _keys_body(locsT_ref, rows_ref, keys_ref):
    ri = pl.program_id(1)
    xs = locsT_ref[0, 0:1, :]
    ys = locsT_ref[0, 1:2, :]
    xr = rows_ref[0, :, 0:1]
    yr = rows_ref[0, :, 1:2]
    dx = xr - xs
    dy = yr - ys
    sq = dx * dx + dy * dy
    rows_g = ri * ROWS + lax.broadcasted_iota(jnp.int32, (ROWS, NPAD), 0)
    cols = lax.broadcasted_iota(jnp.int32, (ROWS, NPAD), 1)
    sq = jnp.where(rows_g == cols, jnp.float32(BIG), sq)
    ikeys = lax.bitcast_convert_type(sq, jnp.int32)
    ikeys = (ikeys & jnp.int32(-1024)) | cols
    keys_ref[...] = lax.bitcast_convert_type(ikeys, jnp.float32)


def _sel_body(keys_hbm, valsq_hbm, dst_hbm, bufA, bufB, osq, odst,
              semA, semB):
    wid = lax.axis_index("s") * 2 + lax.axis_index("c")
    base = jnp.minimum((wid * (HROWS // NW)) // 8 * 8, HROWS - WROWS)

    def start(slab_idx, buf, sem):
        row0 = pl.multiple_of(base + slab_idx * SLAB, 8)
        pltpu.async_copy(keys_hbm.at[pl.ds(row0, SLAB)], buf, sem)

    def drain(buf, sem):
        pltpu.make_async_copy(
            keys_hbm.at[pl.ds(0, SLAB)], buf, sem).wait()

    def process(slab_idx, buf):
        row0 = pl.multiple_of(base + slab_idx * SLAB, 8)

        def row_body(r):
            level = []
            for c in range(NCHUNK):
                v = buf[r, pl.ds(c * 16, 16)]
                level.append(plsc.sort_key_val(v, v)[0])
            while len(level) > 1:
                nxt = []
                for i in range(0, len(level), 2):
                    a = level[i]
                    bm = lax.rev(level[i + 1], (0,))
                    m = jnp.minimum(a, bm)
                    nxt.append(plsc.sort_key_val(m, m)[0])
                level = nxt
            best = level[0]
            ik = plsc.bitcast(best, jnp.int32)
            col = ik & jnp.int32(1023)
            sqv = plsc.bitcast(ik & jnp.int32(-1024), jnp.float32)
            grow = row0 + r
            boff = (grow // 1000) * 1000
            osq[pl.ds(r * KS, KS)] = sqv
            odst[pl.ds(r * KS, KS)] = col + boff

        plsc.parallel_loop(0, SLAB, 1, unroll=2)(row_body)
        pltpu.sync_copy(osq, valsq_hbm.at[pl.ds(row0 * KS, SLAB * KS)])
        pltpu.sync_copy(odst, dst_hbm.at[pl.ds(row0 * KS, SLAB * KS)])

    start(0, bufA, semA)

    def pair_body(q, _):
        start(2 * q + 1, bufB, semB)
        drain(bufA, semA)
        process(2 * q, bufA)
        start(2 * q + 2, bufA, semA)
        drain(bufB, semB)
        process(2 * q + 1, bufB)
        return _

    lax.fori_loop(0, NSLAB // 2 - 1, pair_body, None)
    start(NSLAB - 1, bufB, semB)
    drain(bufA, semA)
    process(NSLAB - 2, bufA)
    drain(bufB, semB)
    process(NSLAB - 1, bufB)


def _emb_body(v1_ref, v2_ref, w_ref, b_ref, out_ref):
    half1 = pl.program_id(0) < (HROWS * KS // (128 * VB))
    v = jnp.where(half1, v1_ref[...], v2_ref[...])
    vt = jnp.transpose(v)
    vt = jnp.sqrt(jnp.maximum(vt, 1e-12))
    w = w_ref[...]
    bb = b_ref[...]
    for j in range(vt.shape[1]):
        out_ref[j * 128:(j + 1) * 128, :] = vt[:, j:j + 1] * w + bb


def _tc_keys_half(locsT_h, locs_h):
    return pl.pallas_call(
        _keys_body,
        grid=(HB, 1000 // ROWS),
        in_specs=[
            pl.BlockSpec((1, 2, NPAD), lambda bi, ri: (bi, 0, 0)),
            pl.BlockSpec((1, ROWS, 2), lambda bi, ri: (bi, ri, 0)),
        ],
        out_specs=pl.BlockSpec((ROWS, NPAD),
                               lambda bi, ri: (bi * (1000 // ROWS) + ri, 0)),
        out_shape=jax.ShapeDtypeStruct((HROWS, NPAD), jnp.float32),
    )(locsT_h, locs_h)


def _sc_sel_half(keys_h):
    mesh = plsc.VectorSubcoreMesh(core_axis_name="c", subcore_axis_name="s")
    sel = functools.partial(
        pl.kernel,
        mesh=mesh,
        out_type=[
            jax.ShapeDtypeStruct((HROWS * KS,), jnp.float32),
            jax.ShapeDtypeStruct((HROWS * KS,), jnp.int32),
        ],
        scratch_types=[
            pltpu.VMEM((SLAB, NPAD), jnp.float32),
            pltpu.VMEM((SLAB, NPAD), jnp.float32),
            pltpu.VMEM((SLAB * KS,), jnp.float32),
            pltpu.VMEM((SLAB * KS,), jnp.int32),
            pltpu.SemaphoreType.DMA,
            pltpu.SemaphoreType.DMA,
        ],
        compiler_params=pltpu.CompilerParams(needs_layout_passes=False),
    )(_sel_body)
    return sel(keys_h)


VB = 200
HSTEPS = HROWS * KS // (128 * VB)


def _tc_emb(valsq1, valsq2, W, b):
    E = 2 * HROWS * KS
    EB = 128 * VB
    EH = HROWS * KS
    return pl.pallas_call(
        _emb_body,
        grid=(E // EB,),
        in_specs=[
            pl.BlockSpec((VB, 128), lambda i: (jnp.minimum(i, HSTEPS - 1), 0)),
            pl.BlockSpec((VB, 128),
                         (lambda i: (jnp.maximum(i - HSTEPS, 0), 0))),
            pl.BlockSpec((1, EMBED), lambda i: (0, 0)),
            pl.BlockSpec((1, EMBED), lambda i: (0, 0)),
        ],
        out_specs=pl.BlockSpec((EB, EMBED), lambda i: (i, 0)),
        out_shape=jax.ShapeDtypeStruct((E, EMBED), jnp.float32),
    )(valsq1.reshape(EH // 128, 128), valsq2.reshape(EH // 128, 128),
      W.reshape(1, EMBED), b.reshape(1, EMBED))


def kernel(locs, init_embedding, W, b):
    B, N, _ = locs.shape
    locsT = jnp.transpose(locs, (0, 2, 1))
    locsT = jnp.pad(locsT, ((0, 0), (0, 0), (0, NPAD - N)),
                    constant_values=1e4)

    keys1 = _tc_keys_half(locsT[:HB], locs[:HB])
    valsq1, dst1 = _sc_sel_half(keys1)
    keys2 = _tc_keys_half(locsT[HB:], locs[HB:])
    valsq2, dst2 = _sc_sel_half(keys2)
    edge_emb = _tc_emb(valsq1, valsq2, W, b)

    offs = (jnp.arange(B) * N)[:, None]
    src = (jnp.repeat(jnp.arange(N), KS)[None, :] + offs).reshape(-1)
    edge_index = jnp.stack([src, jnp.concatenate([dst1, dst2 + HROWS])])
    x = init_embedding.reshape(B * N, EMBED)
    return x, edge_index, edge_emb
